# baseline (speedup 1.0000x reference)
import functools

import jax
import jax.numpy as jnp
from jax import lax
from jax.experimental import pallas as pl
from jax.experimental.pallas import tpu as pltpu
from jax.experimental.pallas import tpu_sc as plsc

_MESH = plsc.VectorSubcoreMesh(core_axis_name="c", subcore_axis_name="s")


@functools.partial(
    pl.kernel,
    out_type=jax.ShapeDtypeStruct((2,), jnp.float32),
    mesh=_MESH,
    scratch_types=[
        pltpu.VMEM((16,), jnp.float32),
        pltpu.VMEM((16,), jnp.float32),
        pltpu.VMEM((16,), jnp.float32),
    ],
)
def _sc_gather(x_hbm, out_hbm, r1_v, r2_v, o_v):
    c = lax.axis_index("c")
    s = lax.axis_index("s")

    @pl.when(jnp.logical_and(c == 0, s == 0))
    def _():
        pltpu.sync_copy(x_hbm.at[1, pl.ds(0, 16)], r1_v)
        pltpu.sync_copy(x_hbm.at[2, pl.ds(0, 16)], r2_v)
        v1 = r1_v[...]
        v2 = r2_v[...]
        a = v1[2]
        b = v2[1]
        lane = lax.iota(jnp.int32, 16)
        o_v[...] = jnp.where(lane == 0, a, b)
        pltpu.sync_copy(o_v.at[pl.ds(0, 2)], out_hbm)


def kernel(x):
    return _sc_gather(x)


# speedup vs baseline: 1.2720x; 1.2720x over previous
import functools

import jax
import jax.numpy as jnp
from jax import lax
from jax.experimental import pallas as pl
from jax.experimental.pallas import tpu as pltpu
from jax.experimental.pallas import tpu_sc as plsc

_MESH = plsc.ScalarSubcoreMesh(axis_name="c", num_cores=1)


@functools.partial(
    pl.kernel,
    out_type=jax.ShapeDtypeStruct((2,), jnp.float32),
    mesh=_MESH,
    scratch_types=[
        pltpu.SMEM((16,), jnp.float32),
        pltpu.SMEM((16,), jnp.float32),
        pltpu.SMEM((2,), jnp.float32),
    ],
)
def _sc_gather(x_hbm, out_hbm, r1_s, r2_s, o_s):
    pltpu.sync_copy(x_hbm.at[1, pl.ds(0, 16)], r1_s)
    pltpu.sync_copy(x_hbm.at[2, pl.ds(0, 16)], r2_s)
    o_s[0] = r1_s[2]
    o_s[1] = r2_s[1]
    pltpu.sync_copy(o_s, out_hbm)


def kernel(x):
    return _sc_gather(x)


# speedup vs baseline: 15.3646x; 12.0789x over previous
import jax
import jax.numpy as jnp
from jax.experimental import pallas as pl


def _gather_kernel(x_ref, o_ref):
    a = x_ref[1, 2]
    b = x_ref[2, 1]
    col = jax.lax.iota(jnp.int32, 2)
    o_ref[...] = jnp.where(col == 0, a, b)


def kernel(x):
    return pl.pallas_call(
        _gather_kernel,
        grid=(1,),
        in_specs=[pl.BlockSpec((8, 128), lambda i: (0, 0))],
        out_specs=pl.BlockSpec((2,), lambda i: (0,)),
        out_shape=jax.ShapeDtypeStruct((2,), jnp.float32),
    )(x)


# speedup vs baseline: 16.2624x; 1.0584x over previous
import jax
import jax.numpy as jnp
from jax.experimental import pallas as pl
from jax.experimental.pallas import tpu as pltpu


def _gather_kernel(x_hbm, o_ref, rows_v, sem):
    cp = pltpu.make_async_copy(x_hbm.at[pl.ds(1, 2), pl.ds(0, 128)], rows_v, sem)
    cp.start()
    cp.wait()
    a = rows_v[0, 2]
    b = rows_v[1, 1]
    col = jax.lax.iota(jnp.int32, 2)
    o_ref[...] = jnp.where(col == 0, a, b)


def kernel(x):
    return pl.pallas_call(
        _gather_kernel,
        grid=(1,),
        in_specs=[pl.BlockSpec(memory_space=pl.ANY)],
        out_specs=pl.BlockSpec((2,), lambda i: (0,)),
        out_shape=jax.ShapeDtypeStruct((2,), jnp.float32),
        scratch_shapes=[
            pltpu.VMEM((2, 128), jnp.float32),
            pltpu.SemaphoreType.DMA,
        ],
    )(x)
